# Initial kernel scaffold; baseline (speedup 1.0000x reference)
import jax
import jax.numpy as jnp
from jax.experimental import pallas as pl


def kernel(x, edge_index, W1_l, W1_r, b1, W2_l, W2_r, b2):
    raise NotImplementedError("write your pallas kernel here")


# speedup vs baseline: 5.8186x; 5.8186x over previous
import functools

import jax
import jax.numpy as jnp
from jax import lax
from jax.experimental import pallas as pl
from jax.experimental.pallas import tpu as pltpu
from jax.experimental.pallas import tpu_sc as plsc

N_NODES = 10000
N_EDGES = 320000
D = 128
NP = 10240
NW = 32
GP = 128
NG = N_EDGES // GP
G_FULL = NG // NW
G_REM = NG - G_FULL * NW
RPT = NP // 16

_MESH = plsc.VectorSubcoreMesh(core_axis_name="c", subcore_axis_name="s")


def _sc_aggregate(xe, src, dst, z2d):
    @functools.partial(
        pl.kernel,
        mesh=_MESH,
        out_type=jax.ShapeDtypeStruct((2, NP, D), jnp.float32),
        scratch_types=[
            pltpu.VMEM((GP,), jnp.int32),
            pltpu.VMEM((GP,), jnp.int32),
            pltpu.VMEM((GP, D), jnp.float32),
            pltpu.VMEM_SHARED((NP, D), jnp.float32),
            pltpu.SemaphoreType.DMA,
        ],
    )
    def agg(xe_hbm, src_hbm, dst_hbm, z2d_hbm, out_hbm, sidx, didx, rows, acc, sem):
        core = lax.axis_index("c")
        tid = lax.axis_index("s")
        w = core * 16 + tid

        pltpu.sync_copy(z2d_hbm, rows)
        rbase = tid * RPT
        for i in range(RPT // GP):
            pltpu.sync_copy(rows, acc.at[pl.ds(rbase + i * GP, GP)])
        plsc.subcore_barrier()

        def do_group(gi):
            base = gi * GP
            pltpu.sync_copy(src_hbm.at[pl.ds(base, GP)], sidx)
            pltpu.sync_copy(dst_hbm.at[pl.ds(base, GP)], didx)
            pltpu.async_copy(xe_hbm.at[sidx], rows, sem).wait()
            pltpu.sync_copy(rows, acc.at[didx], add=True)

        def body(g, carry):
            do_group(g * NW + w)
            return carry

        lax.fori_loop(0, G_FULL, body, 0)

        @pl.when(w < G_REM)
        def _():
            do_group(G_FULL * NW + w)

        plsc.subcore_barrier()

        for i in range(RPT // GP):
            pltpu.sync_copy(acc.at[pl.ds(rbase + i * GP, GP)], rows)
            pltpu.sync_copy(rows, out_hbm.at[core, pl.ds(rbase + i * GP, GP)])

    return agg(xe, src, dst, z2d)


def _sc_count(dst, z2d, o2d):
    @functools.partial(
        pl.kernel,
        mesh=_MESH,
        out_type=jax.ShapeDtypeStruct((2, NP, D), jnp.float32),
        scratch_types=[
            pltpu.VMEM((GP,), jnp.int32),
            pltpu.VMEM((GP, D), jnp.float32),
            pltpu.VMEM_SHARED((NP, D), jnp.float32),
        ],
    )
    def cnt_k(dst_hbm, z2d_hbm, o2d_hbm, out_hbm, didx, rows, acc):
        core = lax.axis_index("c")
        tid = lax.axis_index("s")
        w = core * 16 + tid

        pltpu.sync_copy(z2d_hbm, rows)
        rbase = tid * RPT
        for i in range(RPT // GP):
            pltpu.sync_copy(rows, acc.at[pl.ds(rbase + i * GP, GP)])
        plsc.subcore_barrier()

        pltpu.sync_copy(o2d_hbm, rows)

        def do_group(gi):
            pltpu.sync_copy(dst_hbm.at[pl.ds(gi * GP, GP)], didx)
            pltpu.sync_copy(rows, acc.at[didx], add=True)

        def body(g, carry):
            do_group(g * NW + w)
            return carry

        lax.fori_loop(0, G_FULL, body, 0)

        @pl.when(w < G_REM)
        def _():
            do_group(G_FULL * NW + w)

        plsc.subcore_barrier()

        for i in range(RPT // GP):
            pltpu.sync_copy(acc.at[pl.ds(rbase + i * GP, GP)], rows)
            pltpu.sync_copy(rows, out_hbm.at[core, pl.ds(rbase + i * GP, GP)])

    return cnt_k(dst, z2d, o2d)


def _tc_dense(xe, agg_part, cnt_part, W_l, W_r, b, relu):
    B = 1280

    def body(x_ref, a_ref, c_ref, wl_ref, wr_ref, b_ref, o_ref):
        a = a_ref[0] + a_ref[1]
        c = c_ref[0] + c_ref[1]
        mean = a / jnp.maximum(c, 1.0)
        dn = (((1,), (1,)), ((), ()))
        out = (lax.dot_general(mean, wl_ref[...], dn,
                               preferred_element_type=jnp.float32)
               + lax.dot_general(x_ref[...], wr_ref[...], dn,
                                 preferred_element_type=jnp.float32)
               + b_ref[...][None, :])
        if relu:
            out = jnp.maximum(out, 0.0)
        o_ref[...] = out

    return pl.pallas_call(
        body,
        grid=(NP // B,),
        in_specs=[
            pl.BlockSpec((B, D), lambda i: (i, 0)),
            pl.BlockSpec((2, B, D), lambda i: (0, i, 0)),
            pl.BlockSpec((2, B, D), lambda i: (0, i, 0)),
            pl.BlockSpec((D, D), lambda i: (0, 0)),
            pl.BlockSpec((D, D), lambda i: (0, 0)),
            pl.BlockSpec((D,), lambda i: (0,)),
        ],
        out_specs=pl.BlockSpec((B, D), lambda i: (i, 0)),
        out_shape=jax.ShapeDtypeStruct((NP, D), jnp.float32),
    )(xe, agg_part, cnt_part, W_l, W_r, b)


def kernel(x, edge_index, W1_l, W1_r, b1, W2_l, W2_r, b2):
    src = edge_index[0]
    dst = edge_index[1]

    xe = jnp.pad(x, ((0, NP - N_NODES), (0, 0)))
    z2d = jnp.zeros((GP, D), jnp.float32)
    o2d = jnp.ones((GP, D), jnp.float32)

    cnt = _sc_count(dst, z2d, o2d)
    agg1 = _sc_aggregate(xe, src, dst, z2d)
    h = _tc_dense(xe, agg1, cnt, W1_l, W1_r, b1, relu=True)
    agg2 = _sc_aggregate(h, src, dst, z2d)
    out = _tc_dense(h, agg2, cnt, W2_l, W2_r, b2, relu=False)
    return out[:N_NODES]
